# Initial kernel scaffold; baseline (speedup 1.0000x reference)
import jax
import jax.numpy as jnp
from jax.experimental import pallas as pl


def kernel(node_types, pos, edge_index, batch, W_embed, b_embed, W_enc_s, b_enc_s, W_enc_r, Wq, Wk, Wrk, Wvs, Wrv, Wvv, Wsv, W_dec):
    raise NotImplementedError("write your pallas kernel here")


# speedup vs baseline: 20.8414x; 20.8414x over previous
import functools

import jax
import jax.numpy as jnp
from jax import lax
from jax.experimental import pallas as pl
from jax.experimental.pallas import tpu as pltpu
from jax.experimental.pallas import tpu_sc as plsc

N = 50000
E = 800000
H = 16
NB = 8
RMAX = 2.5

NC = 2
NS = 16
NW = NC * NS
GS = 128
NGS = 5
EPC = GS * NGS
NCHUNK = E // EPC
NROW = E // GS
RPW = N // NS

_mesh = plsc.VectorSubcoreMesh(
    core_axis_name="c", subcore_axis_name="s", num_cores=NC, num_subcores=NS)

_f32 = jnp.float32
_i32 = jnp.int32


def _mm(a, b):
    return lax.dot_general(a, b, (((1,), (0,)), ((), ())),
                           precision=lax.Precision.HIGHEST,
                           preferred_element_type=_f32)


BN = 2000
BE = 4000


def _enc_body(nt, we, be, ws, bs, s_out):
    emb = _mm(nt[...], we[...]) + be[...]
    s_out[...] = _mm(emb, ws[...]) + bs[...]


def _tc_encode(node_types, W_embed, b_embed, W_enc_s, b_enc_s):
    return pl.pallas_call(
        _enc_body,
        grid=(N // BN,),
        in_specs=[
            pl.BlockSpec((BN, 4), lambda i: (i, 0)),
            pl.BlockSpec((4, 64), lambda i: (0, 0)),
            pl.BlockSpec((1, 64), lambda i: (0, 0)),
            pl.BlockSpec((64, 16), lambda i: (0, 0)),
            pl.BlockSpec((1, 16), lambda i: (0, 0)),
        ],
        out_specs=pl.BlockSpec((BN, 16), lambda i: (i, 0)),
        out_shape=jax.ShapeDtypeStruct((N, 16), _f32),
    )(node_types, W_embed, b_embed.reshape(1, 64), W_enc_s,
      b_enc_s.reshape(1, 16))


def _geom_body(gsr, gdr, wenc_w, wrk_w, wrv_w,
               u8_o, wu0_o, wu1_o, wu2_o, rk0_o, rk1_o, rv0_o, rv1_o):
    a = gsr[...]
    b = gdr[...]
    rel = a[:, :3] - b[:, :3]
    rr = jnp.sum(rel * rel, axis=1, keepdims=True) + 1e-9
    r = jnp.sqrt(rr)
    u = rel / r
    u8_o[...] = jnp.concatenate([u, jnp.zeros((BE, 5), _f32)], axis=1)
    centers = lax.broadcasted_iota(_i32, (1, NB), 1).astype(_f32) * (
        RMAX / (NB - 1))
    width = RMAX / NB
    rbf = jnp.exp(-((r - centers) ** 2) / (2.0 * width * width))
    fc = 0.5 * (jnp.cos(jnp.pi * jnp.clip(r / RMAX, 0.0, 1.0)) + 1.0)
    rbf = rbf * fc
    wenc = _mm(rbf, wenc_w[...])
    wu0_o[...] = wenc * u[:, 0:1]
    wu1_o[...] = wenc * u[:, 1:2]
    wu2_o[...] = wenc * u[:, 2:3]
    rk0_o[...] = _mm(rbf, wrk_w[0])
    rk1_o[...] = _mm(rbf, wrk_w[1])
    rv0_o[...] = _mm(rbf, wrv_w[0])
    rv1_o[...] = _mm(rbf, wrv_w[1])


def _tc_geom(gs2, gd2, W_enc_r, Wrk, Wrv):
    eout = jax.ShapeDtypeStruct((E, 16), _f32)
    e16 = pl.BlockSpec((BE, 16), lambda i: (i, 0))
    return pl.pallas_call(
        _geom_body,
        grid=(E // BE,),
        in_specs=[
            pl.BlockSpec((BE, 8), lambda i: (i, 0)),
            pl.BlockSpec((BE, 8), lambda i: (i, 0)),
            pl.BlockSpec((8, 16), lambda i: (0, 0)),
            pl.BlockSpec((2, 8, 16), lambda i: (0, 0, 0)),
            pl.BlockSpec((2, 8, 16), lambda i: (0, 0, 0)),
        ],
        out_specs=[pl.BlockSpec((BE, 8), lambda i: (i, 0)),
                   e16, e16, e16, e16, e16, e16, e16],
        out_shape=[jax.ShapeDtypeStruct((E, 8), _f32),
                   eout, eout, eout, eout, eout, eout, eout],
    )(gs2, gd2, W_enc_r, Wrk, Wrv)


def _tab_body(s_r, v0_r, v1_r, v2_r, wq, wk, wvs, wsv, wvv,
              qt_o, kt_o, t16_o, t0_o, t1_o, t2_o):
    s = s_r[...]
    qt_o[...] = _mm(s, wq[...])
    kt_o[...] = _mm(s, wk[...])
    t16_o[...] = _mm(s, wvs[...])
    sv = _mm(s, wsv[...])
    m0 = _mm(v0_r[...], wvv[...])
    m1 = _mm(v1_r[...], wvv[...])
    m2 = _mm(v2_r[...], wvv[...])
    t0_o[...] = jnp.concatenate([m0, sv], axis=1)
    t1_o[...] = jnp.concatenate([m1, sv], axis=1)
    t2_o[...] = jnp.concatenate([m2, sv], axis=1)


def _tc_tab(s, v0, v1, v2, wq, wk, wvs, wsv, wvv):
    n16 = pl.BlockSpec((BN, 16), lambda i: (i, 0))
    n32 = pl.BlockSpec((BN, 32), lambda i: (i, 0))
    w16 = pl.BlockSpec((16, 16), lambda i: (0, 0))
    o16 = jax.ShapeDtypeStruct((N, 16), _f32)
    o32 = jax.ShapeDtypeStruct((N, 32), _f32)
    return pl.pallas_call(
        _tab_body,
        grid=(N // BN,),
        in_specs=[n16, n16, n16, n16, w16, w16, w16, w16, w16],
        out_specs=[n16, n16, n16, n32, n32, n32],
        out_shape=[o16, o16, o16, o32, o32, o32],
    )(s, v0, v1, v2, wq, wk, wvs, wsv, wvv)


def _lg_body(qd_r, ks_r, rk_r, lg_o, mx_o):
    i = pl.program_id(0)
    p = qd_r[...] * ks_r[...] * rk_r[...]
    lg = jnp.sum(p, axis=1, keepdims=True) * 0.25
    lg_o[...] = lg
    bm = jnp.full((8, 128), jnp.max(lg), _f32)

    @pl.when(i == 0)
    def _():
        mx_o[...] = jnp.full((8, 128), -3e38, _f32)

    mx_o[...] = jnp.maximum(mx_o[...], bm)


def _tc_logits(qd, ks, rk):
    e16 = pl.BlockSpec((BE, 16), lambda i: (i, 0))
    return pl.pallas_call(
        _lg_body,
        grid=(E // BE,),
        in_specs=[e16, e16, e16],
        out_specs=[pl.BlockSpec((BE, 1), lambda i: (i, 0)),
                   pl.BlockSpec((8, 128), lambda i: (0, 0))],
        out_shape=[jax.ShapeDtypeStruct((E, 1), _f32),
                   jax.ShapeDtypeStruct((8, 128), _f32)],
    )(qd, ks, rk)


def _soft_body(lg_r, mx_r, u8_r, rv_r,
               ex_o, exrv_o, ex16_o, exu0_o, exu1_o, exu2_o):
    gmax = jnp.max(mx_r[...])
    ex = jnp.exp(lg_r[...] - gmax)
    ex_o[...] = ex
    exrv_o[...] = ex * rv_r[...]
    ex16 = ex * jnp.ones((1, 16), _f32)
    ex16_o[...] = ex16
    u8 = u8_r[...]
    exu0_o[...] = ex16 * u8[:, 0:1]
    exu1_o[...] = ex16 * u8[:, 1:2]
    exu2_o[...] = ex16 * u8[:, 2:3]


def _tc_soft(lg1, mx, u8, rv):
    e16 = pl.BlockSpec((BE, 16), lambda i: (i, 0))
    eout = jax.ShapeDtypeStruct((E, 16), _f32)
    return pl.pallas_call(
        _soft_body,
        grid=(E // BE,),
        in_specs=[pl.BlockSpec((BE, 1), lambda i: (i, 0)),
                  pl.BlockSpec((8, 128), lambda i: (0, 0)),
                  pl.BlockSpec((BE, 8), lambda i: (i, 0)),
                  e16],
        out_specs=[pl.BlockSpec((BE, 1), lambda i: (i, 0)),
                   e16, e16, e16, e16, e16],
        out_shape=[jax.ShapeDtypeStruct((E, 1), _f32),
                   eout, eout, eout, eout, eout],
    )(lg1, mx, u8, rv)


def _upd_body(s_r, v0_r, v1_r, v2_r, sa_r, va0_r, va1_r, va2_r, den_r,
              sn_o, v0_o, v1_o, v2_o):
    inv = 1.0 / (den_r[...] + 1e-9)
    sn_o[...] = s_r[...] + sa_r[...] * inv
    v0_o[...] = v0_r[...] + va0_r[...] * inv
    v1_o[...] = v1_r[...] + va1_r[...] * inv
    v2_o[...] = v2_r[...] + va2_r[...] * inv


def _tc_upd(s, v0, v1, v2, sacc, va0, va1, va2, den1):
    n16 = pl.BlockSpec((BN, 16), lambda i: (i, 0))
    o16 = jax.ShapeDtypeStruct((N, 16), _f32)
    return pl.pallas_call(
        _upd_body,
        grid=(N // BN,),
        in_specs=[n16] * 8 + [pl.BlockSpec((BN, 1), lambda i: (i, 0))],
        out_specs=[n16] * 4,
        out_shape=[o16] * 4,
    )(s, v0, v1, v2, sacc, va0, va1, va2, den1)


def _out_body(v0_r, v1_r, v2_r, va0_r, va1_r, va2_r, den_r, wd, o_ref):
    inv = 1.0 / (den_r[...] + 1e-9)
    o0 = _mm(v0_r[...] + va0_r[...] * inv, wd[...])
    o1 = _mm(v1_r[...] + va1_r[...] * inv, wd[...])
    o2 = _mm(v2_r[...] + va2_r[...] * inv, wd[...])
    o_ref[...] = jnp.concatenate([o0, o1, o2], axis=1)


def _tc_out(v0, v1, v2, va0, va1, va2, den1, W_dec):
    n16 = pl.BlockSpec((BN, 16), lambda i: (i, 0))
    return pl.pallas_call(
        _out_body,
        grid=(N // BN,),
        in_specs=[n16] * 6 + [pl.BlockSpec((BN, 1), lambda i: (i, 0)),
                              pl.BlockSpec((16, 1), lambda i: (0, 0))],
        out_specs=pl.BlockSpec((BN, 3), lambda i: (i, 0)),
        out_shape=jax.ShapeDtypeStruct((N, 3), _f32),
    )(v0, v1, v2, va0, va1, va2, den1, W_dec)


def _drain(descs):
    for d in descs:
        d.wait()


def _make_dual_gather(w1, w2):
    @functools.partial(
        pl.kernel,
        out_type=[jax.ShapeDtypeStruct((NROW, GS, w1), _f32),
                  jax.ShapeDtypeStruct((NROW, GS, w2), _f32)],
        mesh=_mesh,
        compiler_params=pltpu.CompilerParams(use_tc_tiling_on_sc=False),
        scratch_types=[
            pltpu.VMEM((NGS, GS), _i32),
            pltpu.VMEM((NGS, GS), _i32),
            pltpu.VMEM((NGS, GS, w1), _f32),
            pltpu.VMEM((NGS, GS, w2), _f32),
            pltpu.SemaphoreType.DMA,
        ],
    )
    def k(tab1, tab2, idx1, idx2, o1, o2, i1_v, i2_v, a_v, b_v, sem):
        cid = lax.axis_index("c")
        sid = lax.axis_index("s")
        w = cid * NS + sid
        cnt = (NCHUNK - w + NW - 1) // NW

        def chunk(kk, _):
            c = w + kk * NW
            r5 = pl.ds(c * NGS, NGS)
            _drain([pltpu.async_copy(idx1.at[r5], i1_v, sem),
                    pltpu.async_copy(idx2.at[r5], i2_v, sem)])
            ds = []
            for g in range(NGS):
                ds.append(pltpu.async_copy(tab1.at[i1_v.at[g]], a_v.at[g],
                                           sem))
                ds.append(pltpu.async_copy(tab2.at[i2_v.at[g]], b_v.at[g],
                                           sem))
            _drain(ds)
            _drain([pltpu.async_copy(a_v, o1.at[r5], sem),
                    pltpu.async_copy(b_v, o2.at[r5], sem)])
            return 0

        lax.fori_loop(0, cnt, chunk, 0)

    return k


_sc_gpos = _make_dual_gather(8, 8)
_sc_qk = _make_dual_gather(16, 16)


@functools.partial(
    pl.kernel,
    out_type=[jax.ShapeDtypeStruct((N, 16), _f32),
              jax.ShapeDtypeStruct((N, 16), _f32)],
    mesh=_mesh,
    compiler_params=pltpu.CompilerParams(use_tc_tiling_on_sc=False),
    scratch_types=[
        pltpu.VMEM((NGS, GS), _i32),
        pltpu.VMEM((NGS, GS, 16), _f32),
        pltpu.VMEM_SHARED((N, 16), _f32),
        pltpu.SemaphoreType.DMA,
    ],
)
def _sc_spair(va3, vb3, dst2, z16, a_o, b_o, idxd_v, b1_v, sh2, sem):
    cid = lax.axis_index("c")
    sid = lax.axis_index("s")
    rows = pl.ds(sid * RPW, RPW)
    pltpu.sync_copy(z16.at[rows], sh2.at[rows])
    plsc.subcore_barrier()

    cnt = (NCHUNK - sid + NS - 1) // NS

    def chunk(k, _):
        c = sid + k * NS
        r5 = pl.ds(c * NGS, NGS)

        @pl.when(cid == 0)
        def _():
            _drain([pltpu.async_copy(dst2.at[r5], idxd_v, sem),
                    pltpu.async_copy(va3.at[r5], b1_v, sem)])

        @pl.when(cid == 1)
        def _():
            _drain([pltpu.async_copy(dst2.at[r5], idxd_v, sem),
                    pltpu.async_copy(vb3.at[r5], b1_v, sem)])

        for g in range(NGS):
            pltpu.sync_copy(b1_v.at[g], sh2.at[idxd_v.at[g]], add=True)
        return 0

    lax.fori_loop(0, cnt, chunk, 0)
    plsc.subcore_barrier()

    @pl.when(cid == 0)
    def _():
        pltpu.sync_copy(sh2.at[rows], a_o.at[rows])

    @pl.when(cid == 1)
    def _():
        pltpu.sync_copy(sh2.at[rows], b_o.at[rows])


@functools.partial(
    pl.kernel,
    out_type=[jax.ShapeDtypeStruct((N,), _f32),
              jax.ShapeDtypeStruct((N, 16), _f32),
              jax.ShapeDtypeStruct((N, 16), _f32)],
    mesh=_mesh,
    compiler_params=pltpu.CompilerParams(use_tc_tiling_on_sc=False),
    scratch_types=[
        pltpu.VMEM((NGS, GS), _i32),
        pltpu.VMEM((NGS, GS), _i32),
        pltpu.VMEM((NGS, GS), _f32),
        pltpu.VMEM((NGS, GS, 32), _f32),
        pltpu.VMEM((NGS, GS, 16), _f32),
        pltpu.VMEM((NGS, GS, 16), _f32),
        pltpu.VMEM((NGS, GS, 16), _f32),
        pltpu.VMEM_SHARED((N,), _f32),
        pltpu.VMEM_SHARED((N, 16), _f32),
        pltpu.SemaphoreType.DMA,
    ],
)
def _sc_b1(ex2, t16, t032, exrv3, ex163, exu03, src2, dst2, z1, z16,
           den_o, sacc_o, v0_o,
           idxs_v, idxd_v, ex_v, gt_v, p1_v, p2_v, b1_v, sh1, sh2, sem):
    cid = lax.axis_index("c")
    sid = lax.axis_index("s")
    rows = pl.ds(sid * RPW, RPW)
    pltpu.sync_copy(z16.at[rows], sh2.at[rows])

    @pl.when(jnp.logical_and(cid == 0, sid == 0))
    def _():
        pltpu.sync_copy(z1, sh1)

    plsc.subcore_barrier()

    cnt = (NCHUNK - sid + NS - 1) // NS

    def chunk(k, _):
        c = sid + k * NS
        r5 = pl.ds(c * NGS, NGS)

        @pl.when(cid == 0)
        def _():
            _drain([pltpu.async_copy(src2.at[r5], idxs_v, sem),
                    pltpu.async_copy(dst2.at[r5], idxd_v, sem),
                    pltpu.async_copy(ex2.at[r5], ex_v, sem),
                    pltpu.async_copy(exrv3.at[r5], p1_v, sem)])
            ds = [pltpu.async_copy(t16.at[idxs_v.at[g]], p2_v.at[g], sem)
                  for g in range(NGS)]
            _drain(ds)
            for g in range(NGS):
                def eb(i, _c):
                    b1_v[g, i, :] = p2_v[g, i, :] * p1_v[g, i, :]
                    return 0
                lax.fori_loop(0, GS, eb, 0)
            for g in range(NGS):
                pltpu.sync_copy(ex_v.at[g], sh1.at[idxd_v.at[g]], add=True)
                pltpu.sync_copy(b1_v.at[g], sh2.at[idxd_v.at[g]], add=True)

        @pl.when(cid == 1)
        def _():
            _drain([pltpu.async_copy(src2.at[r5], idxs_v, sem),
                    pltpu.async_copy(dst2.at[r5], idxd_v, sem),
                    pltpu.async_copy(ex163.at[r5], p1_v, sem),
                    pltpu.async_copy(exu03.at[r5], p2_v, sem)])
            ds = [pltpu.async_copy(t032.at[idxs_v.at[g]], gt_v.at[g], sem)
                  for g in range(NGS)]
            _drain(ds)
            for g in range(NGS):
                def eb(i, _c):
                    b1_v[g, i, :] = (gt_v[g, i, 0:16] * p1_v[g, i, :]
                                     + gt_v[g, i, 16:32] * p2_v[g, i, :])
                    return 0
                lax.fori_loop(0, GS, eb, 0)
            for g in range(NGS):
                pltpu.sync_copy(b1_v.at[g], sh2.at[idxd_v.at[g]], add=True)
        return 0

    lax.fori_loop(0, cnt, chunk, 0)
    plsc.subcore_barrier()

    @pl.when(cid == 0)
    def _():
        pltpu.sync_copy(sh2.at[rows], sacc_o.at[rows])

        @pl.when(sid == 0)
        def _():
            pltpu.sync_copy(sh1, den_o)

    @pl.when(cid == 1)
    def _():
        pltpu.sync_copy(sh2.at[rows], v0_o.at[rows])


@functools.partial(
    pl.kernel,
    out_type=[jax.ShapeDtypeStruct((N, 16), _f32),
              jax.ShapeDtypeStruct((N, 16), _f32)],
    mesh=_mesh,
    compiler_params=pltpu.CompilerParams(use_tc_tiling_on_sc=False),
    scratch_types=[
        pltpu.VMEM((NGS, GS), _i32),
        pltpu.VMEM((NGS, GS), _i32),
        pltpu.VMEM((NGS, GS, 32), _f32),
        pltpu.VMEM((NGS, GS, 16), _f32),
        pltpu.VMEM((NGS, GS, 16), _f32),
        pltpu.VMEM((NGS, GS, 16), _f32),
        pltpu.VMEM_SHARED((N, 16), _f32),
        pltpu.SemaphoreType.DMA,
    ],
)
def _sc_b2(t132, t232, ex163, exu13, exu23, src2, dst2, z16,
           v1_o, v2_o,
           idxs_v, idxd_v, gt_v, p1_v, p2_v, b1_v, sh2, sem):
    cid = lax.axis_index("c")
    sid = lax.axis_index("s")
    rows = pl.ds(sid * RPW, RPW)
    pltpu.sync_copy(z16.at[rows], sh2.at[rows])
    plsc.subcore_barrier()

    cnt = (NCHUNK - sid + NS - 1) // NS

    def chunk(k, _):
        c = sid + k * NS
        r5 = pl.ds(c * NGS, NGS)

        @pl.when(cid == 0)
        def _():
            _drain([pltpu.async_copy(src2.at[r5], idxs_v, sem),
                    pltpu.async_copy(dst2.at[r5], idxd_v, sem),
                    pltpu.async_copy(ex163.at[r5], p1_v, sem),
                    pltpu.async_copy(exu13.at[r5], p2_v, sem)])
            ds = [pltpu.async_copy(t132.at[idxs_v.at[g]], gt_v.at[g], sem)
                  for g in range(NGS)]
            _drain(ds)

        @pl.when(cid == 1)
        def _():
            _drain([pltpu.async_copy(src2.at[r5], idxs_v, sem),
                    pltpu.async_copy(dst2.at[r5], idxd_v, sem),
                    pltpu.async_copy(ex163.at[r5], p1_v, sem),
                    pltpu.async_copy(exu23.at[r5], p2_v, sem)])
            ds = [pltpu.async_copy(t232.at[idxs_v.at[g]], gt_v.at[g], sem)
                  for g in range(NGS)]
            _drain(ds)

        for g in range(NGS):
            def eb(i, _c):
                b1_v[g, i, :] = (gt_v[g, i, 0:16] * p1_v[g, i, :]
                                 + gt_v[g, i, 16:32] * p2_v[g, i, :])
                return 0
            lax.fori_loop(0, GS, eb, 0)
        for g in range(NGS):
            pltpu.sync_copy(b1_v.at[g], sh2.at[idxd_v.at[g]], add=True)
        return 0

    lax.fori_loop(0, cnt, chunk, 0)
    plsc.subcore_barrier()

    @pl.when(cid == 0)
    def _():
        pltpu.sync_copy(sh2.at[rows], v1_o.at[rows])

    @pl.when(cid == 1)
    def _():
        pltpu.sync_copy(sh2.at[rows], v2_o.at[rows])


def kernel(node_types, pos, edge_index, batch, W_embed, b_embed, W_enc_s,
           b_enc_s, W_enc_r, Wq, Wk, Wrk, Wvs, Wrv, Wvv, Wsv, W_dec):
    del batch
    src2 = edge_index[0].reshape(NROW, GS)
    dst2 = edge_index[1].reshape(NROW, GS)
    pos8 = jnp.concatenate([pos, jnp.zeros((N, 5), _f32)], axis=1)
    z1 = jnp.zeros((N,), _f32)
    z16 = jnp.zeros((N, 16), _f32)

    s = _tc_encode(node_types, W_embed, b_embed, W_enc_s, b_enc_s)

    gs3, gd3 = _sc_gpos(pos8, pos8, src2, dst2)
    u8, wu0, wu1, wu2, rk0, rk1, rv0, rv1 = _tc_geom(
        gs3.reshape(E, 8), gd3.reshape(E, 8), W_enc_r, Wrk, Wrv)
    rk = (rk0, rk1)
    rv = (rv0, rv1)

    r3 = lambda a: a.reshape(NROW, GS, 16)
    v0, v1 = _sc_spair(r3(wu0), r3(wu1), dst2, z16)
    v2, _unused = _sc_spair(r3(wu2), r3(wu2), dst2, z16)

    for l in range(2):
        qt, kt, t16, t0, t1, t2 = _tc_tab(s, v0, v1, v2, Wq[l], Wk[l],
                                          Wvs[l], Wsv[l], Wvv[l])
        qd3, ks3 = _sc_qk(qt, kt, dst2, src2)
        lg1, mx = _tc_logits(qd3.reshape(E, 16), ks3.reshape(E, 16), rk[l])
        ex1, exrv, ex16, exu0, exu1, exu2 = _tc_soft(lg1, mx, u8, rv[l])
        den, sacc, va0 = _sc_b1(
            ex1.reshape(NROW, GS), t16, t0, r3(exrv), r3(ex16), r3(exu0),
            src2, dst2, z1, z16)
        va1, va2 = _sc_b2(t1, t2, r3(ex16), r3(exu1), r3(exu2), src2, dst2,
                          z16)
        den1 = den.reshape(N, 1)
        if l == 0:
            s, v0, v1, v2 = _tc_upd(s, v0, v1, v2, sacc, va0, va1, va2, den1)
        else:
            return _tc_out(v0, v1, v2, va0, va1, va2, den1, W_dec)


# speedup vs baseline: 62.8222x; 3.0143x over previous
import functools

import jax
import jax.numpy as jnp
from jax import lax
from jax.experimental import pallas as pl
from jax.experimental.pallas import tpu as pltpu
from jax.experimental.pallas import tpu_sc as plsc

N = 50000
E = 800000
H = 16
NB = 8
RMAX = 2.5

NC = 2
NS = 16
NW = NC * NS
GS = 128
NGS = 5
EPC = GS * NGS
NCHUNK = E // EPC
NROW = E // GS
RPW = N // NS
EPR = E * 16 // 128

_mesh = plsc.VectorSubcoreMesh(
    core_axis_name="c", subcore_axis_name="s", num_cores=NC, num_subcores=NS)

_f32 = jnp.float32
_i32 = jnp.int32


def _mm(a, b):
    return lax.dot_general(a, b, (((1,), (0,)), ((), ())),
                           precision=lax.Precision.HIGHEST,
                           preferred_element_type=_f32)


BN = 2000
BR = 2000

_pk = pl.BlockSpec((BR, 128), lambda i: (i, 0))
_bd128 = pl.BlockSpec((128, 128), lambda i: (0, 0))
_pkout = jax.ShapeDtypeStruct((EPR, 128), _f32)


def _enc_body(nt, we, be, ws, bs, s_out):
    emb = _mm(nt[...], we[...]) + be[...]
    s_out[...] = _mm(emb, ws[...]) + bs[...]


def _tc_encode(node_types, W_embed, b_embed, W_enc_s, b_enc_s):
    return pl.pallas_call(
        _enc_body,
        grid=(N // BN,),
        in_specs=[
            pl.BlockSpec((BN, 4), lambda i: (i, 0)),
            pl.BlockSpec((4, 64), lambda i: (0, 0)),
            pl.BlockSpec((1, 64), lambda i: (0, 0)),
            pl.BlockSpec((64, 16), lambda i: (0, 0)),
            pl.BlockSpec((1, 16), lambda i: (0, 0)),
        ],
        out_specs=pl.BlockSpec((BN, 16), lambda i: (i, 0)),
        out_shape=jax.ShapeDtypeStruct((N, 16), _f32),
    )(node_types, W_embed, b_embed.reshape(1, 64), W_enc_s,
      b_enc_s.reshape(1, 16))


def _geom_body(gsr, gdr, bdsum, bdenc, bdrk0, bdrk1, bdrv0, bdrv1,
               bdp0, bdp1, bdp2,
               u_o, wu0_o, wu1_o, wu2_o, rk0_o, rk1_o, rv0_o, rv1_o):
    rel = gsr[...] - gdr[...]
    rr = _mm(rel * rel, bdsum[...]) + 1e-9
    rinv = lax.rsqrt(rr)
    r = rr * rinv
    u = rel * rinv
    u_o[...] = u
    li = lax.broadcasted_iota(_i32, (1, 128), 1)
    b = li % 16
    cen = b.astype(_f32) * (RMAX / (NB - 1))
    msk = jnp.where(b < NB, 1.0, 0.0).astype(_f32)
    width = RMAX / NB
    fc = 0.5 * (jnp.cos(jnp.pi * jnp.clip(r / RMAX, 0.0, 1.0)) + 1.0)
    rbf = jnp.exp(-((r - cen) ** 2) * (1.0 / (2.0 * width * width)))
    rbf = rbf * fc * msk
    wenc = _mm(rbf, bdenc[...])
    wu0_o[...] = wenc * _mm(u, bdp0[...])
    wu1_o[...] = wenc * _mm(u, bdp1[...])
    wu2_o[...] = wenc * _mm(u, bdp2[...])
    rk0_o[...] = _mm(rbf, bdrk0[...])
    rk1_o[...] = _mm(rbf, bdrk1[...])
    rv0_o[...] = _mm(rbf, bdrv0[...])
    rv1_o[...] = _mm(rbf, bdrv1[...])


def _tc_geom(gsP, gdP, bdsum, bdenc, bdrk0, bdrk1, bdrv0, bdrv1,
             bdp0, bdp1, bdp2):
    return pl.pallas_call(
        _geom_body,
        grid=(EPR // BR,),
        in_specs=[_pk, _pk] + [_bd128] * 6 + [_bd128] * 3,
        out_specs=[_pk] * 8,
        out_shape=[_pkout] * 8,
    )(gsP, gdP, bdsum, bdenc, bdrk0, bdrk1, bdrv0, bdrv1, bdp0, bdp1, bdp2)


def _tab_body(s_r, v0_r, v1_r, v2_r, wq, wk, wvs, wsv, wvv,
              qt_o, kt_o, t16_o, t0_o, t1_o, t2_o):
    s = s_r[...]
    qt_o[...] = _mm(s, wq[...])
    kt_o[...] = _mm(s, wk[...])
    t16_o[...] = _mm(s, wvs[...])
    sv = _mm(s, wsv[...])
    m0 = _mm(v0_r[...], wvv[...])
    m1 = _mm(v1_r[...], wvv[...])
    m2 = _mm(v2_r[...], wvv[...])
    t0_o[...] = jnp.concatenate([m0, sv], axis=1)
    t1_o[...] = jnp.concatenate([m1, sv], axis=1)
    t2_o[...] = jnp.concatenate([m2, sv], axis=1)


def _tc_tab(s, v0, v1, v2, wq, wk, wvs, wsv, wvv):
    n16 = pl.BlockSpec((BN, 16), lambda i: (i, 0))
    n32 = pl.BlockSpec((BN, 32), lambda i: (i, 0))
    w16 = pl.BlockSpec((16, 16), lambda i: (0, 0))
    o16 = jax.ShapeDtypeStruct((N, 16), _f32)
    o32 = jax.ShapeDtypeStruct((N, 32), _f32)
    return pl.pallas_call(
        _tab_body,
        grid=(N // BN,),
        in_specs=[n16, n16, n16, n16, w16, w16, w16, w16, w16],
        out_specs=[n16, n16, n16, n32, n32, n32],
        out_shape=[o16, o16, o16, o32, o32, o32],
    )(s, v0, v1, v2, wq, wk, wvs, wsv, wvv)


def _lg_body(qd_r, ks_r, rk_r, bdsum, lg_o, mx_o):
    i = pl.program_id(0)
    p = qd_r[...] * ks_r[...] * rk_r[...]
    lg = _mm(p, bdsum[...]) * 0.25
    lg_o[...] = lg
    bm = jnp.full((8, 128), jnp.max(lg), _f32)

    @pl.when(i == 0)
    def _():
        mx_o[...] = jnp.full((8, 128), -3e38, _f32)

    mx_o[...] = jnp.maximum(mx_o[...], bm)


def _tc_logits(qdP, ksP, rkP, bdsum):
    return pl.pallas_call(
        _lg_body,
        grid=(EPR // BR,),
        in_specs=[_pk, _pk, _pk, _bd128],
        out_specs=[_pk, pl.BlockSpec((8, 128), lambda i: (0, 0))],
        out_shape=[_pkout, jax.ShapeDtypeStruct((8, 128), _f32)],
    )(qdP, ksP, rkP, bdsum)


def _soft_body(lg_r, mx_r, u_r, rv_r, bdp0, bdp1, bdp2,
               ex_o, exrv_o, exu0_o, exu1_o, exu2_o):
    gmax = jnp.max(mx_r[...])
    ex = jnp.exp(lg_r[...] - gmax)
    ex_o[...] = ex
    exrv_o[...] = ex * rv_r[...]
    u = u_r[...]
    exu0_o[...] = ex * _mm(u, bdp0[...])
    exu1_o[...] = ex * _mm(u, bdp1[...])
    exu2_o[...] = ex * _mm(u, bdp2[...])


def _tc_soft(lgP, mx, uP, rvP, bdp0, bdp1, bdp2):
    return pl.pallas_call(
        _soft_body,
        grid=(EPR // BR,),
        in_specs=[_pk, pl.BlockSpec((8, 128), lambda i: (0, 0)),
                  _pk, _pk, _bd128, _bd128, _bd128],
        out_specs=[_pk] * 5,
        out_shape=[_pkout] * 5,
    )(lgP, mx, uP, rvP, bdp0, bdp1, bdp2)


def _upd_body(s_r, v0_r, v1_r, v2_r, sa_r, sb_r, va0_r, va1_r, va2_r,
              den_r, sn_o, v0_o, v1_o, v2_o):
    inv = 1.0 / (den_r[...][:, 0:1] + 1e-9)
    sn_o[...] = s_r[...] + (sa_r[...] + sb_r[...]) * inv
    v0_o[...] = v0_r[...] + va0_r[...] * inv
    v1_o[...] = v1_r[...] + va1_r[...] * inv
    v2_o[...] = v2_r[...] + va2_r[...] * inv


def _tc_upd(s, v0, v1, v2, sacc_a, sacc_b, va0, va1, va2, den16):
    n16 = pl.BlockSpec((BN, 16), lambda i: (i, 0))
    o16 = jax.ShapeDtypeStruct((N, 16), _f32)
    return pl.pallas_call(
        _upd_body,
        grid=(N // BN,),
        in_specs=[n16] * 10,
        out_specs=[n16] * 4,
        out_shape=[o16] * 4,
    )(s, v0, v1, v2, sacc_a, sacc_b, va0, va1, va2, den16)


def _out_body(v0_r, v1_r, v2_r, va0_r, va1_r, va2_r, den_r, wd, o_ref):
    inv = 1.0 / (den_r[...][:, 0:1] + 1e-9)
    o0 = _mm(v0_r[...] + va0_r[...] * inv, wd[...])
    o1 = _mm(v1_r[...] + va1_r[...] * inv, wd[...])
    o2 = _mm(v2_r[...] + va2_r[...] * inv, wd[...])
    o_ref[...] = jnp.concatenate([o0, o1, o2], axis=1)


def _tc_out(v0, v1, v2, va0, va1, va2, den16, W_dec):
    n16 = pl.BlockSpec((BN, 16), lambda i: (i, 0))
    return pl.pallas_call(
        _out_body,
        grid=(N // BN,),
        in_specs=[n16] * 7 + [pl.BlockSpec((16, 1), lambda i: (0, 0))],
        out_specs=pl.BlockSpec((BN, 3), lambda i: (i, 0)),
        out_shape=jax.ShapeDtypeStruct((N, 3), _f32),
    )(v0, v1, v2, va0, va1, va2, den16, W_dec)


def _drain(descs):
    for d in descs:
        d.wait()


@functools.partial(
    pl.kernel,
    out_type=[jax.ShapeDtypeStruct((NROW, GS, 16), _f32),
              jax.ShapeDtypeStruct((NROW, GS, 16), _f32)],
    mesh=_mesh,
    compiler_params=pltpu.CompilerParams(use_tc_tiling_on_sc=False),
    scratch_types=[
        pltpu.VMEM((NGS, GS), _i32),
        pltpu.VMEM((NGS, GS), _i32),
        pltpu.VMEM((NGS, GS, 16), _f32),
        pltpu.VMEM((NGS, GS, 16), _f32),
        pltpu.SemaphoreType.DMA,
    ],
)
def _sc_dualg(tab1, tab2, idx1, idx2, o1, o2, i1_v, i2_v, a_v, b_v, sem):
    cid = lax.axis_index("c")
    sid = lax.axis_index("s")
    w = cid * NS + sid
    cnt = (NCHUNK - w + NW - 1) // NW

    def chunk(kk, _):
        c = w + kk * NW
        r5 = pl.ds(c * NGS, NGS)
        _drain([pltpu.async_copy(idx1.at[r5], i1_v, sem),
                pltpu.async_copy(idx2.at[r5], i2_v, sem)])
        ds = []
        for g in range(NGS):
            ds.append(pltpu.async_copy(tab1.at[i1_v.at[g]], a_v.at[g], sem))
            ds.append(pltpu.async_copy(tab2.at[i2_v.at[g]], b_v.at[g], sem))
        _drain(ds)
        _drain([pltpu.async_copy(a_v, o1.at[r5], sem),
                pltpu.async_copy(b_v, o2.at[r5], sem)])
        return 0

    lax.fori_loop(0, cnt, chunk, 0)


@functools.partial(
    pl.kernel,
    out_type=[jax.ShapeDtypeStruct((N, 16), _f32),
              jax.ShapeDtypeStruct((N, 16), _f32)],
    mesh=_mesh,
    compiler_params=pltpu.CompilerParams(use_tc_tiling_on_sc=False),
    scratch_types=[
        pltpu.VMEM((NGS, GS), _i32),
        pltpu.VMEM((NGS, GS, 16), _f32),
        pltpu.VMEM_SHARED((N, 16), _f32),
        pltpu.SemaphoreType.DMA,
    ],
)
def _sc_spair(va3, vb3, dst2, z16, a_o, b_o, idxd_v, b1_v, sh2, sem):
    cid = lax.axis_index("c")
    sid = lax.axis_index("s")
    rows = pl.ds(sid * RPW, RPW)
    pltpu.sync_copy(z16.at[rows], sh2.at[rows])
    plsc.subcore_barrier()

    cnt = (NCHUNK - sid + NS - 1) // NS

    def chunk(k, _):
        c = sid + k * NS
        r5 = pl.ds(c * NGS, NGS)

        @pl.when(cid == 0)
        def _():
            _drain([pltpu.async_copy(dst2.at[r5], idxd_v, sem),
                    pltpu.async_copy(va3.at[r5], b1_v, sem)])

        @pl.when(cid == 1)
        def _():
            _drain([pltpu.async_copy(dst2.at[r5], idxd_v, sem),
                    pltpu.async_copy(vb3.at[r5], b1_v, sem)])

        for g in range(NGS):
            pltpu.sync_copy(b1_v.at[g], sh2.at[idxd_v.at[g]], add=True)
        return 0

    lax.fori_loop(0, cnt, chunk, 0)
    plsc.subcore_barrier()

    @pl.when(cid == 0)
    def _():
        pltpu.sync_copy(sh2.at[rows], a_o.at[rows])

    @pl.when(cid == 1)
    def _():
        pltpu.sync_copy(sh2.at[rows], b_o.at[rows])


@functools.partial(
    pl.kernel,
    out_type=[jax.ShapeDtypeStruct((N, 16), _f32),
              jax.ShapeDtypeStruct((N, 16), _f32)],
    mesh=_mesh,
    compiler_params=pltpu.CompilerParams(use_tc_tiling_on_sc=False),
    scratch_types=[
        pltpu.VMEM((NGS, GS), _i32),
        pltpu.VMEM((NGS, GS), _i32),
        pltpu.VMEM((NGS, GS, 16), _f32),
        pltpu.VMEM((NGS, GS, 16), _f32),
        pltpu.VMEM((NGS, GS, 16), _f32),
        pltpu.VMEM_SHARED((N, 16), _f32),
        pltpu.SemaphoreType.DMA,
    ],
)
def _sc_b1(t16, exrv3, src2, dst2, z16,
           sacc_o, dup_o,
           idxs_v, idxd_v, gt_v, p1_v, b1_v, sh2, sem):
    cid = lax.axis_index("c")
    sid = lax.axis_index("s")
    rows = pl.ds(sid * RPW, RPW)
    pltpu.sync_copy(z16.at[rows], sh2.at[rows])
    plsc.subcore_barrier()

    w = cid * NS + sid
    cnt = (NCHUNK - w + NW - 1) // NW

    def chunk(k, _):
        c = w + k * NW
        r5 = pl.ds(c * NGS, NGS)
        _drain([pltpu.async_copy(src2.at[r5], idxs_v, sem),
                pltpu.async_copy(dst2.at[r5], idxd_v, sem),
                pltpu.async_copy(exrv3.at[r5], p1_v, sem)])
        ds = [pltpu.async_copy(t16.at[idxs_v.at[g]], gt_v.at[g], sem)
              for g in range(NGS)]
        _drain(ds)
        for g in range(NGS):
            @plsc.parallel_loop(0, GS, unroll=8)
            def _(i):
                b1_v[g, i, :] = gt_v[g, i, :] * p1_v[g, i, :]
        for g in range(NGS):
            pltpu.sync_copy(b1_v.at[g], sh2.at[idxd_v.at[g]], add=True)
        return 0

    lax.fori_loop(0, cnt, chunk, 0)
    plsc.subcore_barrier()

    @pl.when(cid == 0)
    def _():
        pltpu.sync_copy(sh2.at[rows], sacc_o.at[rows])

    @pl.when(cid == 1)
    def _():
        pltpu.sync_copy(sh2.at[rows], dup_o.at[rows])


@functools.partial(
    pl.kernel,
    out_type=[jax.ShapeDtypeStruct((N, 16), _f32),
              jax.ShapeDtypeStruct((N, 16), _f32)],
    mesh=_mesh,
    compiler_params=pltpu.CompilerParams(use_tc_tiling_on_sc=False),
    scratch_types=[
        pltpu.VMEM((NGS, GS), _i32),
        pltpu.VMEM((NGS, GS), _i32),
        pltpu.VMEM((NGS, GS, 32), _f32),
        pltpu.VMEM((NGS, GS, 16), _f32),
        pltpu.VMEM((NGS, GS, 16), _f32),
        pltpu.VMEM((NGS, GS, 16), _f32),
        pltpu.VMEM_SHARED((N, 16), _f32),
        pltpu.SemaphoreType.DMA,
    ],
)
def _sc_b2(ta32, tb32, ex3, exua3, exub3, src2, dst2, z16,
           va_o, vb_o,
           idxs_v, idxd_v, gt_v, p1_v, p2_v, b1_v, sh2, sem):
    cid = lax.axis_index("c")
    sid = lax.axis_index("s")
    rows = pl.ds(sid * RPW, RPW)
    pltpu.sync_copy(z16.at[rows], sh2.at[rows])
    plsc.subcore_barrier()

    cnt = (NCHUNK - sid + NS - 1) // NS

    def chunk(k, _):
        c = sid + k * NS
        r5 = pl.ds(c * NGS, NGS)

        @pl.when(cid == 0)
        def _():
            _drain([pltpu.async_copy(src2.at[r5], idxs_v, sem),
                    pltpu.async_copy(dst2.at[r5], idxd_v, sem),
                    pltpu.async_copy(ex3.at[r5], p1_v, sem),
                    pltpu.async_copy(exua3.at[r5], p2_v, sem)])
            ds = [pltpu.async_copy(ta32.at[idxs_v.at[g]], gt_v.at[g], sem)
                  for g in range(NGS)]
            _drain(ds)

        @pl.when(cid == 1)
        def _():
            _drain([pltpu.async_copy(src2.at[r5], idxs_v, sem),
                    pltpu.async_copy(dst2.at[r5], idxd_v, sem),
                    pltpu.async_copy(ex3.at[r5], p1_v, sem),
                    pltpu.async_copy(exub3.at[r5], p2_v, sem)])
            ds = [pltpu.async_copy(tb32.at[idxs_v.at[g]], gt_v.at[g], sem)
                  for g in range(NGS)]
            _drain(ds)

        for g in range(NGS):
            @plsc.parallel_loop(0, GS, unroll=8)
            def _(i):
                b1_v[g, i, :] = (gt_v[g, i, 0:16] * p1_v[g, i, :]
                                 + gt_v[g, i, 16:32] * p2_v[g, i, :])
        for g in range(NGS):
            pltpu.sync_copy(b1_v.at[g], sh2.at[idxd_v.at[g]], add=True)
        return 0

    lax.fori_loop(0, cnt, chunk, 0)
    plsc.subcore_barrier()

    @pl.when(cid == 0)
    def _():
        pltpu.sync_copy(sh2.at[rows], va_o.at[rows])

    @pl.when(cid == 1)
    def _():
        pltpu.sync_copy(sh2.at[rows], vb_o.at[rows])


def kernel(node_types, pos, edge_index, batch, W_embed, b_embed, W_enc_s,
           b_enc_s, W_enc_r, Wq, Wk, Wrk, Wvs, Wrv, Wvv, Wsv, W_dec):
    del batch
    src2 = edge_index[0].reshape(NROW, GS)
    dst2 = edge_index[1].reshape(NROW, GS)
    pos16 = jnp.concatenate([pos, jnp.zeros((N, 13), _f32)], axis=1)
    z16 = jnp.zeros((N, 16), _f32)

    eye8 = jnp.eye(8, dtype=_f32)
    bdsum = jnp.kron(eye8, jnp.ones((16, 16), _f32))
    pad8 = jnp.zeros((8, 16), _f32)
    bdenc = jnp.kron(eye8, jnp.concatenate([W_enc_r, pad8], axis=0))
    bdrk = [jnp.kron(eye8, jnp.concatenate([Wrk[l], pad8], axis=0))
            for l in range(2)]
    bdrv = [jnp.kron(eye8, jnp.concatenate([Wrv[l], pad8], axis=0))
            for l in range(2)]
    bdp = [jnp.kron(eye8, jnp.zeros((16, 16), _f32).at[d].set(1.0))
           for d in range(3)]

    r3 = lambda a: a.reshape(NROW, GS, 16)
    rp = lambda a: a.reshape(EPR, 128)

    s = _tc_encode(node_types, W_embed, b_embed, W_enc_s, b_enc_s)

    gs3, gd3 = _sc_dualg(pos16, pos16, src2, dst2)
    uP, wu0, wu1, wu2, rk0, rk1, rv0, rv1 = _tc_geom(
        rp(gs3), rp(gd3), bdsum, bdenc, bdrk[0], bdrk[1], bdrv[0], bdrv[1],
        bdp[0], bdp[1], bdp[2])
    rk = (rk0, rk1)
    rv = (rv0, rv1)

    v0, v1 = _sc_spair(r3(wu0), r3(wu1), dst2, z16)
    v2, _x = _sc_spair(r3(wu2), r3(wu2), dst2, z16)

    for l in range(2):
        qt, kt, t16, t0, t1, t2 = _tc_tab(s, v0, v1, v2, Wq[l], Wk[l],
                                          Wvs[l], Wsv[l], Wvv[l])
        qd3, ks3 = _sc_dualg(qt, kt, dst2, src2)
        lgP, mx = _tc_logits(rp(qd3), rp(ks3), rk[l], bdsum)
        exP, exrv, exu0, exu1, exu2 = _tc_soft(lgP, mx, uP, rv[l],
                                               bdp[0], bdp[1], bdp[2])
        den16, _d = _sc_spair(r3(exP), r3(exP), dst2, z16)
        va0, va1 = _sc_b2(t0, t1, r3(exP), r3(exu0), r3(exu1), src2, dst2,
                          z16)
        va2, _v2b = _sc_b2(t2, t2, r3(exP), r3(exu2), r3(exu2), src2, dst2,
                           z16)
        if l == 0:
            sacc_a, sacc_b = _sc_b1(t16, r3(exrv), src2, dst2, z16)
            s, v0, v1, v2 = _tc_upd(s, v0, v1, v2, sacc_a, sacc_b,
                                    va0, va1, va2, den16)
        else:
            return _tc_out(v0, v1, v2, va0, va1, va2, den16, W_dec)


# speedup vs baseline: 65.1869x; 1.0376x over previous
import functools

import jax
import jax.numpy as jnp
from jax import lax
from jax.experimental import pallas as pl
from jax.experimental.pallas import tpu as pltpu
from jax.experimental.pallas import tpu_sc as plsc

N = 50000
E = 800000
H = 16
NB = 8
RMAX = 2.5

NC = 2
NS = 16
NW = NC * NS
GS = 128
NGS = 5
EPC = GS * NGS
NCHUNK = E // EPC
NROW = E // GS
RPW = N // NS
EPR = E * 16 // 128

_mesh = plsc.VectorSubcoreMesh(
    core_axis_name="c", subcore_axis_name="s", num_cores=NC, num_subcores=NS)

_f32 = jnp.float32
_i32 = jnp.int32


def _mm(a, b):
    return lax.dot_general(a, b, (((1,), (0,)), ((), ())),
                           precision=lax.Precision.HIGHEST,
                           preferred_element_type=_f32)


BN = 2000
BR = 2000

_pk = pl.BlockSpec((BR, 128), lambda i: (i, 0))
_bd128 = pl.BlockSpec((128, 128), lambda i: (0, 0))
_pkout = jax.ShapeDtypeStruct((EPR, 128), _f32)


def _enc_body(nt, we, be, ws, bs, s_out):
    emb = _mm(nt[...], we[...]) + be[...]
    s_out[...] = _mm(emb, ws[...]) + bs[...]


def _tc_encode(node_types, W_embed, b_embed, W_enc_s, b_enc_s):
    return pl.pallas_call(
        _enc_body,
        grid=(N // BN,),
        in_specs=[
            pl.BlockSpec((BN, 4), lambda i: (i, 0)),
            pl.BlockSpec((4, 64), lambda i: (0, 0)),
            pl.BlockSpec((1, 64), lambda i: (0, 0)),
            pl.BlockSpec((64, 16), lambda i: (0, 0)),
            pl.BlockSpec((1, 16), lambda i: (0, 0)),
        ],
        out_specs=pl.BlockSpec((BN, 16), lambda i: (i, 0)),
        out_shape=jax.ShapeDtypeStruct((N, 16), _f32),
    )(node_types, W_embed, b_embed.reshape(1, 64), W_enc_s,
      b_enc_s.reshape(1, 16))


def _geom_body(gsr, gdr, bdsum, bdenc, bdrk0, bdrk1, bdrv0, bdrv1,
               bdp0, bdp1, bdp2,
               u_o, wu0_o, wu1_o, wu2_o, rk0_o, rk1_o, rv0_o, rv1_o):
    rel = gsr[...] - gdr[...]
    rr = _mm(rel * rel, bdsum[...]) + 1e-9
    rinv = lax.rsqrt(rr)
    r = rr * rinv
    u = rel * rinv
    u_o[...] = u
    li = lax.broadcasted_iota(_i32, (1, 128), 1)
    b = li % 16
    cen = b.astype(_f32) * (RMAX / (NB - 1))
    msk = jnp.where(b < NB, 1.0, 0.0).astype(_f32)
    width = RMAX / NB
    fc = 0.5 * (jnp.cos(jnp.pi * jnp.clip(r / RMAX, 0.0, 1.0)) + 1.0)
    rbf = jnp.exp(-((r - cen) ** 2) * (1.0 / (2.0 * width * width)))
    rbf = rbf * fc * msk
    wenc = _mm(rbf, bdenc[...])
    wu0_o[...] = wenc * _mm(u, bdp0[...])
    wu1_o[...] = wenc * _mm(u, bdp1[...])
    wu2_o[...] = wenc * _mm(u, bdp2[...])
    rk0_o[...] = _mm(rbf, bdrk0[...])
    rk1_o[...] = _mm(rbf, bdrk1[...])
    rv0_o[...] = _mm(rbf, bdrv0[...])
    rv1_o[...] = _mm(rbf, bdrv1[...])


def _tc_geom(gsP, gdP, bdsum, bdenc, bdrk0, bdrk1, bdrv0, bdrv1,
             bdp0, bdp1, bdp2):
    return pl.pallas_call(
        _geom_body,
        grid=(EPR // BR,),
        in_specs=[_pk, _pk] + [_bd128] * 6 + [_bd128] * 3,
        out_specs=[_pk] * 8,
        out_shape=[_pkout] * 8,
    )(gsP, gdP, bdsum, bdenc, bdrk0, bdrk1, bdrv0, bdrv1, bdp0, bdp1, bdp2)


def _tab_body(s_r, v0_r, v1_r, v2_r, wq, wk, wvs, wsv, wvv,
              qt_o, kt_o, t16_o, t0_o, t1_o, t2_o):
    s = s_r[...]
    qt_o[...] = _mm(s, wq[...])
    kt_o[...] = _mm(s, wk[...])
    t16_o[...] = _mm(s, wvs[...])
    sv = _mm(s, wsv[...])
    m0 = _mm(v0_r[...], wvv[...])
    m1 = _mm(v1_r[...], wvv[...])
    m2 = _mm(v2_r[...], wvv[...])
    t0_o[...] = jnp.concatenate([m0, sv], axis=1)
    t1_o[...] = jnp.concatenate([m1, sv], axis=1)
    t2_o[...] = jnp.concatenate([m2, sv], axis=1)


def _tc_tab(s, v0, v1, v2, wq, wk, wvs, wsv, wvv):
    n16 = pl.BlockSpec((BN, 16), lambda i: (i, 0))
    n32 = pl.BlockSpec((BN, 32), lambda i: (i, 0))
    w16 = pl.BlockSpec((16, 16), lambda i: (0, 0))
    o16 = jax.ShapeDtypeStruct((N, 16), _f32)
    o32 = jax.ShapeDtypeStruct((N, 32), _f32)
    return pl.pallas_call(
        _tab_body,
        grid=(N // BN,),
        in_specs=[n16, n16, n16, n16, w16, w16, w16, w16, w16],
        out_specs=[n16, n16, n16, n32, n32, n32],
        out_shape=[o16, o16, o16, o32, o32, o32],
    )(s, v0, v1, v2, wq, wk, wvs, wsv, wvv)


def _lg_body(qd_r, ks_r, rk_r, bdsum, lg_o, mx_o):
    i = pl.program_id(0)
    p = qd_r[...] * ks_r[...] * rk_r[...]
    lg = _mm(p, bdsum[...]) * 0.25
    lg_o[...] = lg
    bm = jnp.full((8, 128), jnp.max(lg), _f32)

    @pl.when(i == 0)
    def _():
        mx_o[...] = jnp.full((8, 128), -3e38, _f32)

    mx_o[...] = jnp.maximum(mx_o[...], bm)


def _tc_logits(qdP, ksP, rkP, bdsum):
    return pl.pallas_call(
        _lg_body,
        grid=(EPR // BR,),
        in_specs=[_pk, _pk, _pk, _bd128],
        out_specs=[_pk, pl.BlockSpec((8, 128), lambda i: (0, 0))],
        out_shape=[_pkout, jax.ShapeDtypeStruct((8, 128), _f32)],
    )(qdP, ksP, rkP, bdsum)


def _soft_body(lg_r, mx_r, u_r, rv_r, bdp0, bdp1, bdp2,
               ex_o, exrv_o, exu0_o, exu1_o, exu2_o):
    gmax = jnp.max(mx_r[...])
    ex = jnp.exp(lg_r[...] - gmax)
    ex_o[...] = ex
    exrv_o[...] = ex * rv_r[...]
    u = u_r[...]
    exu0_o[...] = ex * _mm(u, bdp0[...])
    exu1_o[...] = ex * _mm(u, bdp1[...])
    exu2_o[...] = ex * _mm(u, bdp2[...])


def _tc_soft(lgP, mx, uP, rvP, bdp0, bdp1, bdp2):
    return pl.pallas_call(
        _soft_body,
        grid=(EPR // BR,),
        in_specs=[_pk, pl.BlockSpec((8, 128), lambda i: (0, 0)),
                  _pk, _pk, _bd128, _bd128, _bd128],
        out_specs=[_pk] * 5,
        out_shape=[_pkout] * 5,
    )(lgP, mx, uP, rvP, bdp0, bdp1, bdp2)


def _upd_body(s_r, v0_r, v1_r, v2_r, sa_r, sb_r, va0_r, va1_r, va2_r,
              den_r, sn_o, v0_o, v1_o, v2_o):
    inv = 1.0 / (den_r[...][:, 0:1] + 1e-9)
    sn_o[...] = s_r[...] + (sa_r[...] + sb_r[...]) * inv
    v0_o[...] = v0_r[...] + va0_r[...] * inv
    v1_o[...] = v1_r[...] + va1_r[...] * inv
    v2_o[...] = v2_r[...] + va2_r[...] * inv


def _tc_upd(s, v0, v1, v2, sacc_a, sacc_b, va0, va1, va2, den16):
    n16 = pl.BlockSpec((BN, 16), lambda i: (i, 0))
    o16 = jax.ShapeDtypeStruct((N, 16), _f32)
    return pl.pallas_call(
        _upd_body,
        grid=(N // BN,),
        in_specs=[n16] * 10,
        out_specs=[n16] * 4,
        out_shape=[o16] * 4,
    )(s, v0, v1, v2, sacc_a, sacc_b, va0, va1, va2, den16)


def _out_body(v0_r, v1_r, v2_r, va0_r, va1_r, va2_r, den_r, wd, o_ref):
    inv = 1.0 / (den_r[...][:, 0:1] + 1e-9)
    o0 = _mm(v0_r[...] + va0_r[...] * inv, wd[...])
    o1 = _mm(v1_r[...] + va1_r[...] * inv, wd[...])
    o2 = _mm(v2_r[...] + va2_r[...] * inv, wd[...])
    o_ref[...] = jnp.concatenate([o0, o1, o2], axis=1)


def _tc_out(v0, v1, v2, va0, va1, va2, den16, W_dec):
    n16 = pl.BlockSpec((BN, 16), lambda i: (i, 0))
    return pl.pallas_call(
        _out_body,
        grid=(N // BN,),
        in_specs=[n16] * 7 + [pl.BlockSpec((16, 1), lambda i: (0, 0))],
        out_specs=pl.BlockSpec((BN, 3), lambda i: (i, 0)),
        out_shape=jax.ShapeDtypeStruct((N, 3), _f32),
    )(v0, v1, v2, va0, va1, va2, den16, W_dec)


def _drain(descs):
    for d in descs:
        d.wait()


@functools.partial(
    pl.kernel,
    out_type=[jax.ShapeDtypeStruct((NROW, GS, 16), _f32),
              jax.ShapeDtypeStruct((NROW, GS, 16), _f32)],
    mesh=_mesh,
    compiler_params=pltpu.CompilerParams(use_tc_tiling_on_sc=False),
    scratch_types=[
        pltpu.VMEM((NGS, GS), _i32),
        pltpu.VMEM((NGS, GS), _i32),
        pltpu.VMEM((NGS, GS, 16), _f32),
        pltpu.VMEM((NGS, GS, 16), _f32),
        pltpu.SemaphoreType.DMA,
    ],
)
def _sc_dualg(tab1, tab2, idx1, idx2, o1, o2, i1_v, i2_v, a_v, b_v, sem):
    cid = lax.axis_index("c")
    sid = lax.axis_index("s")
    w = cid * NS + sid
    cnt = (NCHUNK - w + NW - 1) // NW

    def chunk(kk, _):
        c = w + kk * NW
        r5 = pl.ds(c * NGS, NGS)
        _drain([pltpu.async_copy(idx1.at[r5], i1_v, sem),
                pltpu.async_copy(idx2.at[r5], i2_v, sem)])
        ds = []
        for g in range(NGS):
            ds.append(pltpu.async_copy(tab1.at[i1_v.at[g]], a_v.at[g], sem))
            ds.append(pltpu.async_copy(tab2.at[i2_v.at[g]], b_v.at[g], sem))
        _drain(ds)
        _drain([pltpu.async_copy(a_v, o1.at[r5], sem),
                pltpu.async_copy(b_v, o2.at[r5], sem)])
        return 0

    lax.fori_loop(0, cnt, chunk, 0)


@functools.partial(
    pl.kernel,
    out_type=[jax.ShapeDtypeStruct((N, 16), _f32),
              jax.ShapeDtypeStruct((N, 16), _f32)],
    mesh=_mesh,
    compiler_params=pltpu.CompilerParams(use_tc_tiling_on_sc=False),
    scratch_types=[
        pltpu.VMEM((NGS, GS), _i32),
        pltpu.VMEM((NGS, GS, 16), _f32),
        pltpu.VMEM_SHARED((N, 16), _f32),
        pltpu.SemaphoreType.DMA,
    ],
)
def _sc_spair(va3, vb3, dst2, z16, a_o, b_o, idxd_v, b1_v, sh2, sem):
    cid = lax.axis_index("c")
    sid = lax.axis_index("s")
    rows = pl.ds(sid * RPW, RPW)
    pltpu.sync_copy(z16.at[rows], sh2.at[rows])
    plsc.subcore_barrier()

    cnt = (NCHUNK - sid + NS - 1) // NS

    def chunk(k, _):
        c = sid + k * NS
        r5 = pl.ds(c * NGS, NGS)

        @pl.when(cid == 0)
        def _():
            _drain([pltpu.async_copy(dst2.at[r5], idxd_v, sem),
                    pltpu.async_copy(va3.at[r5], b1_v, sem)])

        @pl.when(cid == 1)
        def _():
            _drain([pltpu.async_copy(dst2.at[r5], idxd_v, sem),
                    pltpu.async_copy(vb3.at[r5], b1_v, sem)])

        _drain([pltpu.async_copy(b1_v.at[g], sh2.at[idxd_v.at[g]], sem,
                                 add=True) for g in range(NGS)])
        return 0

    lax.fori_loop(0, cnt, chunk, 0)
    plsc.subcore_barrier()

    @pl.when(cid == 0)
    def _():
        pltpu.sync_copy(sh2.at[rows], a_o.at[rows])

    @pl.when(cid == 1)
    def _():
        pltpu.sync_copy(sh2.at[rows], b_o.at[rows])


@functools.partial(
    pl.kernel,
    out_type=[jax.ShapeDtypeStruct((N, 16), _f32),
              jax.ShapeDtypeStruct((N, 16), _f32)],
    mesh=_mesh,
    compiler_params=pltpu.CompilerParams(use_tc_tiling_on_sc=False),
    scratch_types=[
        pltpu.VMEM((NGS, GS), _i32),
        pltpu.VMEM((NGS, GS), _i32),
        pltpu.VMEM((NGS, GS, 16), _f32),
        pltpu.VMEM((NGS, GS, 16), _f32),
        pltpu.VMEM((NGS, GS, 16), _f32),
        pltpu.VMEM_SHARED((N, 16), _f32),
        pltpu.SemaphoreType.DMA,
    ],
)
def _sc_b1(t16, exrv3, src2, dst2, z16,
           sacc_o, dup_o,
           idxs_v, idxd_v, gt_v, p1_v, b1_v, sh2, sem):
    cid = lax.axis_index("c")
    sid = lax.axis_index("s")
    rows = pl.ds(sid * RPW, RPW)
    pltpu.sync_copy(z16.at[rows], sh2.at[rows])
    plsc.subcore_barrier()

    w = cid * NS + sid
    cnt = (NCHUNK - w + NW - 1) // NW

    def chunk(k, _):
        c = w + k * NW
        r5 = pl.ds(c * NGS, NGS)
        _drain([pltpu.async_copy(src2.at[r5], idxs_v, sem),
                pltpu.async_copy(dst2.at[r5], idxd_v, sem),
                pltpu.async_copy(exrv3.at[r5], p1_v, sem)])
        ds = [pltpu.async_copy(t16.at[idxs_v.at[g]], gt_v.at[g], sem)
              for g in range(NGS)]
        _drain(ds)
        for g in range(NGS):
            @plsc.parallel_loop(0, GS, unroll=8)
            def _(i):
                b1_v[g, i, :] = gt_v[g, i, :] * p1_v[g, i, :]
        _drain([pltpu.async_copy(b1_v.at[g], sh2.at[idxd_v.at[g]], sem,
                                 add=True) for g in range(NGS)])
        return 0

    lax.fori_loop(0, cnt, chunk, 0)
    plsc.subcore_barrier()

    @pl.when(cid == 0)
    def _():
        pltpu.sync_copy(sh2.at[rows], sacc_o.at[rows])

    @pl.when(cid == 1)
    def _():
        pltpu.sync_copy(sh2.at[rows], dup_o.at[rows])


@functools.partial(
    pl.kernel,
    out_type=[jax.ShapeDtypeStruct((N, 16), _f32),
              jax.ShapeDtypeStruct((N, 16), _f32)],
    mesh=_mesh,
    compiler_params=pltpu.CompilerParams(use_tc_tiling_on_sc=False),
    scratch_types=[
        pltpu.VMEM((NGS, GS), _i32),
        pltpu.VMEM((NGS, GS), _i32),
        pltpu.VMEM((NGS, GS, 32), _f32),
        pltpu.VMEM((NGS, GS, 16), _f32),
        pltpu.VMEM((NGS, GS, 16), _f32),
        pltpu.VMEM((NGS, GS, 16), _f32),
        pltpu.VMEM_SHARED((N, 16), _f32),
        pltpu.SemaphoreType.DMA,
    ],
)
def _sc_b2(ta32, tb32, ex3, exua3, exub3, src2, dst2, z16,
           va_o, vb_o,
           idxs_v, idxd_v, gt_v, p1_v, p2_v, b1_v, sh2, sem):
    cid = lax.axis_index("c")
    sid = lax.axis_index("s")
    rows = pl.ds(sid * RPW, RPW)
    pltpu.sync_copy(z16.at[rows], sh2.at[rows])
    plsc.subcore_barrier()

    cnt = (NCHUNK - sid + NS - 1) // NS

    def chunk(k, _):
        c = sid + k * NS
        r5 = pl.ds(c * NGS, NGS)

        @pl.when(cid == 0)
        def _():
            _drain([pltpu.async_copy(src2.at[r5], idxs_v, sem),
                    pltpu.async_copy(dst2.at[r5], idxd_v, sem),
                    pltpu.async_copy(ex3.at[r5], p1_v, sem),
                    pltpu.async_copy(exua3.at[r5], p2_v, sem)])
            ds = [pltpu.async_copy(ta32.at[idxs_v.at[g]], gt_v.at[g], sem)
                  for g in range(NGS)]
            _drain(ds)

        @pl.when(cid == 1)
        def _():
            _drain([pltpu.async_copy(src2.at[r5], idxs_v, sem),
                    pltpu.async_copy(dst2.at[r5], idxd_v, sem),
                    pltpu.async_copy(ex3.at[r5], p1_v, sem),
                    pltpu.async_copy(exub3.at[r5], p2_v, sem)])
            ds = [pltpu.async_copy(tb32.at[idxs_v.at[g]], gt_v.at[g], sem)
                  for g in range(NGS)]
            _drain(ds)

        for g in range(NGS):
            @plsc.parallel_loop(0, GS, unroll=8)
            def _(i):
                b1_v[g, i, :] = (gt_v[g, i, 0:16] * p1_v[g, i, :]
                                 + gt_v[g, i, 16:32] * p2_v[g, i, :])
        _drain([pltpu.async_copy(b1_v.at[g], sh2.at[idxd_v.at[g]], sem,
                                 add=True) for g in range(NGS)])
        return 0

    lax.fori_loop(0, cnt, chunk, 0)
    plsc.subcore_barrier()

    @pl.when(cid == 0)
    def _():
        pltpu.sync_copy(sh2.at[rows], va_o.at[rows])

    @pl.when(cid == 1)
    def _():
        pltpu.sync_copy(sh2.at[rows], vb_o.at[rows])


def kernel(node_types, pos, edge_index, batch, W_embed, b_embed, W_enc_s,
           b_enc_s, W_enc_r, Wq, Wk, Wrk, Wvs, Wrv, Wvv, Wsv, W_dec):
    del batch
    src2 = edge_index[0].reshape(NROW, GS)
    dst2 = edge_index[1].reshape(NROW, GS)
    pos16 = jnp.concatenate([pos, jnp.zeros((N, 13), _f32)], axis=1)
    z16 = jnp.zeros((N, 16), _f32)

    eye8 = jnp.eye(8, dtype=_f32)
    bdsum = jnp.kron(eye8, jnp.ones((16, 16), _f32))
    pad8 = jnp.zeros((8, 16), _f32)
    bdenc = jnp.kron(eye8, jnp.concatenate([W_enc_r, pad8], axis=0))
    bdrk = [jnp.kron(eye8, jnp.concatenate([Wrk[l], pad8], axis=0))
            for l in range(2)]
    bdrv = [jnp.kron(eye8, jnp.concatenate([Wrv[l], pad8], axis=0))
            for l in range(2)]
    bdp = [jnp.kron(eye8, jnp.zeros((16, 16), _f32).at[d].set(1.0))
           for d in range(3)]

    r3 = lambda a: a.reshape(NROW, GS, 16)
    rp = lambda a: a.reshape(EPR, 128)

    s = _tc_encode(node_types, W_embed, b_embed, W_enc_s, b_enc_s)

    gs3, gd3 = _sc_dualg(pos16, pos16, src2, dst2)
    uP, wu0, wu1, wu2, rk0, rk1, rv0, rv1 = _tc_geom(
        rp(gs3), rp(gd3), bdsum, bdenc, bdrk[0], bdrk[1], bdrv[0], bdrv[1],
        bdp[0], bdp[1], bdp[2])
    rk = (rk0, rk1)
    rv = (rv0, rv1)

    v0, v1 = _sc_spair(r3(wu0), r3(wu1), dst2, z16)
    v2, _x = _sc_spair(r3(wu2), r3(wu2), dst2, z16)

    for l in range(2):
        qt, kt, t16, t0, t1, t2 = _tc_tab(s, v0, v1, v2, Wq[l], Wk[l],
                                          Wvs[l], Wsv[l], Wvv[l])
        qd3, ks3 = _sc_dualg(qt, kt, dst2, src2)
        lgP, mx = _tc_logits(rp(qd3), rp(ks3), rk[l], bdsum)
        exP, exrv, exu0, exu1, exu2 = _tc_soft(lgP, mx, uP, rv[l],
                                               bdp[0], bdp[1], bdp[2])
        den16, _d = _sc_spair(r3(exP), r3(exP), dst2, z16)
        va0, va1 = _sc_b2(t0, t1, r3(exP), r3(exu0), r3(exu1), src2, dst2,
                          z16)
        va2, _v2b = _sc_b2(t2, t2, r3(exP), r3(exu2), r3(exu2), src2, dst2,
                           z16)
        if l == 0:
            sacc_a, sacc_b = _sc_b1(t16, r3(exrv), src2, dst2, z16)
            s, v0, v1, v2 = _tc_upd(s, v0, v1, v2, sacc_a, sacc_b,
                                    va0, va1, va2, den16)
        else:
            return _tc_out(v0, v1, v2, va0, va1, va2, den16, W_dec)


# speedup vs baseline: 70.2729x; 1.0780x over previous
import functools

import jax
import jax.numpy as jnp
from jax import lax
from jax.experimental import pallas as pl
from jax.experimental.pallas import tpu as pltpu
from jax.experimental.pallas import tpu_sc as plsc

N = 50000
E = 800000
H = 16
NB = 8
RMAX = 2.5

NC = 2
NS = 16
NW = NC * NS
GS = 128
NGS = 5
EPC = GS * NGS
NCHUNK = E // EPC
NROW = E // GS
RPW = N // NS
EPR = E * 16 // 128

_mesh = plsc.VectorSubcoreMesh(
    core_axis_name="c", subcore_axis_name="s", num_cores=NC, num_subcores=NS)

_f32 = jnp.float32
_i32 = jnp.int32


def _mm(a, b):
    return lax.dot_general(a, b, (((1,), (0,)), ((), ())),
                           precision=lax.Precision.HIGHEST,
                           preferred_element_type=_f32)


BN = 2000
BR = 2000

_pk = pl.BlockSpec((BR, 128), lambda i: (i, 0))
_bd128 = pl.BlockSpec((128, 128), lambda i: (0, 0))
_pkout = jax.ShapeDtypeStruct((EPR, 128), _f32)


def _enc_body(nt, we, be, ws, bs, s_out):
    emb = _mm(nt[...], we[...]) + be[...]
    s_out[...] = _mm(emb, ws[...]) + bs[...]


def _tc_encode(node_types, W_embed, b_embed, W_enc_s, b_enc_s):
    return pl.pallas_call(
        _enc_body,
        grid=(N // BN,),
        in_specs=[
            pl.BlockSpec((BN, 4), lambda i: (i, 0)),
            pl.BlockSpec((4, 64), lambda i: (0, 0)),
            pl.BlockSpec((1, 64), lambda i: (0, 0)),
            pl.BlockSpec((64, 16), lambda i: (0, 0)),
            pl.BlockSpec((1, 16), lambda i: (0, 0)),
        ],
        out_specs=pl.BlockSpec((BN, 16), lambda i: (i, 0)),
        out_shape=jax.ShapeDtypeStruct((N, 16), _f32),
    )(node_types, W_embed, b_embed.reshape(1, 64), W_enc_s,
      b_enc_s.reshape(1, 16))


def _geom_body(gsr, gdr, bdsum, bdenc, bdrk0, bdrk1, bdrv0, bdrv1,
               bdp0, bdp1, bdp2,
               u_o, wu0_o, wu1_o, wu2_o, rk0_o, rk1_o, rv0_o, rv1_o):
    rel = gsr[...] - gdr[...]
    rr = _mm(rel * rel, bdsum[...]) + 1e-9
    rinv = lax.rsqrt(rr)
    r = rr * rinv
    u = rel * rinv
    u_o[...] = u
    li = lax.broadcasted_iota(_i32, (1, 128), 1)
    b = li % 16
    cen = b.astype(_f32) * (RMAX / (NB - 1))
    msk = jnp.where(b < NB, 1.0, 0.0).astype(_f32)
    width = RMAX / NB
    fc = 0.5 * (jnp.cos(jnp.pi * jnp.clip(r / RMAX, 0.0, 1.0)) + 1.0)
    rbf = jnp.exp(-((r - cen) ** 2) * (1.0 / (2.0 * width * width)))
    rbf = rbf * fc * msk
    wenc = _mm(rbf, bdenc[...])
    wu0_o[...] = wenc * _mm(u, bdp0[...])
    wu1_o[...] = wenc * _mm(u, bdp1[...])
    wu2_o[...] = wenc * _mm(u, bdp2[...])
    rk0_o[...] = _mm(rbf, bdrk0[...])
    rk1_o[...] = _mm(rbf, bdrk1[...])
    rv0_o[...] = _mm(rbf, bdrv0[...])
    rv1_o[...] = _mm(rbf, bdrv1[...])


def _tc_geom(gsP, gdP, bdsum, bdenc, bdrk0, bdrk1, bdrv0, bdrv1,
             bdp0, bdp1, bdp2):
    return pl.pallas_call(
        _geom_body,
        grid=(EPR // BR,),
        in_specs=[_pk, _pk] + [_bd128] * 6 + [_bd128] * 3,
        out_specs=[_pk] * 8,
        out_shape=[_pkout] * 8,
    )(gsP, gdP, bdsum, bdenc, bdrk0, bdrk1, bdrv0, bdrv1, bdp0, bdp1, bdp2)


def _tab_body(s_r, v0_r, v1_r, v2_r, wq, wk, wvs, wsv, wvv,
              qt_o, kt_o, t16_o, t0_o, t1_o, t2_o):
    s = s_r[...]
    qt_o[...] = _mm(s, wq[...])
    kt_o[...] = _mm(s, wk[...])
    t16_o[...] = _mm(s, wvs[...])
    sv = _mm(s, wsv[...])
    m0 = _mm(v0_r[...], wvv[...])
    m1 = _mm(v1_r[...], wvv[...])
    m2 = _mm(v2_r[...], wvv[...])
    t0_o[...] = jnp.concatenate([m0, sv], axis=1)
    t1_o[...] = jnp.concatenate([m1, sv], axis=1)
    t2_o[...] = jnp.concatenate([m2, sv], axis=1)


def _tc_tab(s, v0, v1, v2, wq, wk, wvs, wsv, wvv):
    n16 = pl.BlockSpec((BN, 16), lambda i: (i, 0))
    n32 = pl.BlockSpec((BN, 32), lambda i: (i, 0))
    w16 = pl.BlockSpec((16, 16), lambda i: (0, 0))
    o16 = jax.ShapeDtypeStruct((N, 16), _f32)
    o32 = jax.ShapeDtypeStruct((N, 32), _f32)
    return pl.pallas_call(
        _tab_body,
        grid=(N // BN,),
        in_specs=[n16, n16, n16, n16, w16, w16, w16, w16, w16],
        out_specs=[n16, n16, n16, n32, n32, n32],
        out_shape=[o16, o16, o16, o32, o32, o32],
    )(s, v0, v1, v2, wq, wk, wvs, wsv, wvv)


def _add3_body(a0, b0, a1, b1, a2, b2, v0_o, v1_o, v2_o):
    v0_o[...] = a0[...] + b0[...]
    v1_o[...] = a1[...] + b1[...]
    v2_o[...] = a2[...] + b2[...]


def _tc_add3(a0, b0, a1, b1, a2, b2):
    n16 = pl.BlockSpec((BN, 16), lambda i: (i, 0))
    o16 = jax.ShapeDtypeStruct((N, 16), _f32)
    return pl.pallas_call(
        _add3_body,
        grid=(N // BN,),
        in_specs=[n16] * 6,
        out_specs=[n16] * 3,
        out_shape=[o16] * 3,
    )(a0, b0, a1, b1, a2, b2)


def _lg_body(qd_r, ks_r, rk_r, bdsum, lg_o, mx_o):
    i = pl.program_id(0)
    p = qd_r[...] * ks_r[...] * rk_r[...]
    lg = _mm(p, bdsum[...]) * 0.25
    lg_o[...] = lg
    bm = jnp.full((8, 128), jnp.max(lg), _f32)

    @pl.when(i == 0)
    def _():
        mx_o[...] = jnp.full((8, 128), -3e38, _f32)

    mx_o[...] = jnp.maximum(mx_o[...], bm)


def _tc_logits(qdP, ksP, rkP, bdsum):
    return pl.pallas_call(
        _lg_body,
        grid=(EPR // BR,),
        in_specs=[_pk, _pk, _pk, _bd128],
        out_specs=[_pk, pl.BlockSpec((8, 128), lambda i: (0, 0))],
        out_shape=[_pkout, jax.ShapeDtypeStruct((8, 128), _f32)],
    )(qdP, ksP, rkP, bdsum)


def _soft_body(lg_r, mx_r, u_r, rv_r, bdp0, bdp1, bdp2,
               ex_o, exrv_o, exu0_o, exu1_o, exu2_o):
    gmax = jnp.max(mx_r[...])
    ex = jnp.exp(lg_r[...] - gmax)
    ex_o[...] = ex
    exrv_o[...] = ex * rv_r[...]
    u = u_r[...]
    exu0_o[...] = ex * _mm(u, bdp0[...])
    exu1_o[...] = ex * _mm(u, bdp1[...])
    exu2_o[...] = ex * _mm(u, bdp2[...])


def _tc_soft(lgP, mx, uP, rvP, bdp0, bdp1, bdp2):
    return pl.pallas_call(
        _soft_body,
        grid=(EPR // BR,),
        in_specs=[_pk, pl.BlockSpec((8, 128), lambda i: (0, 0)),
                  _pk, _pk, _bd128, _bd128, _bd128],
        out_specs=[_pk] * 5,
        out_shape=[_pkout] * 5,
    )(lgP, mx, uP, rvP, bdp0, bdp1, bdp2)


def _upd_body(s_r, v0_r, v1_r, v2_r, sa_r, sb_r, va0a, va0b, va1a, va1b,
              va2a, va2b, dena, denb, sn_o, v0_o, v1_o, v2_o):
    inv = 1.0 / ((dena[...][:, 0:1] + denb[...][:, 0:1]) + 1e-9)
    sn_o[...] = s_r[...] + (sa_r[...] + sb_r[...]) * inv
    v0_o[...] = v0_r[...] + (va0a[...] + va0b[...]) * inv
    v1_o[...] = v1_r[...] + (va1a[...] + va1b[...]) * inv
    v2_o[...] = v2_r[...] + (va2a[...] + va2b[...]) * inv


def _tc_upd(s, v0, v1, v2, sacc_a, sacc_b, vap, denp):
    n16 = pl.BlockSpec((BN, 16), lambda i: (i, 0))
    o16 = jax.ShapeDtypeStruct((N, 16), _f32)
    return pl.pallas_call(
        _upd_body,
        grid=(N // BN,),
        in_specs=[n16] * 14,
        out_specs=[n16] * 4,
        out_shape=[o16] * 4,
    )(s, v0, v1, v2, sacc_a, sacc_b, vap[0][0], vap[0][1], vap[1][0],
      vap[1][1], vap[2][0], vap[2][1], denp[0], denp[1])


def _out_body(v0_r, v1_r, v2_r, va0a, va0b, va1a, va1b, va2a, va2b,
              dena, denb, wd, o_ref):
    inv = 1.0 / ((dena[...][:, 0:1] + denb[...][:, 0:1]) + 1e-9)
    o0 = _mm(v0_r[...] + (va0a[...] + va0b[...]) * inv, wd[...])
    o1 = _mm(v1_r[...] + (va1a[...] + va1b[...]) * inv, wd[...])
    o2 = _mm(v2_r[...] + (va2a[...] + va2b[...]) * inv, wd[...])
    o_ref[...] = jnp.concatenate([o0, o1, o2], axis=1)


def _tc_out(v0, v1, v2, vap, denp, W_dec):
    n16 = pl.BlockSpec((BN, 16), lambda i: (i, 0))
    return pl.pallas_call(
        _out_body,
        grid=(N // BN,),
        in_specs=[n16] * 11 + [pl.BlockSpec((16, 1), lambda i: (0, 0))],
        out_specs=pl.BlockSpec((BN, 3), lambda i: (i, 0)),
        out_shape=jax.ShapeDtypeStruct((N, 3), _f32),
    )(v0, v1, v2, vap[0][0], vap[0][1], vap[1][0], vap[1][1], vap[2][0],
      vap[2][1], denp[0], denp[1], W_dec)


def _drain(descs):
    for d in descs:
        d.wait()


@functools.partial(
    pl.kernel,
    out_type=[jax.ShapeDtypeStruct((NROW, GS, 16), _f32),
              jax.ShapeDtypeStruct((NROW, GS, 16), _f32)],
    mesh=_mesh,
    compiler_params=pltpu.CompilerParams(use_tc_tiling_on_sc=False),
    scratch_types=[
        pltpu.VMEM((NGS, GS), _i32),
        pltpu.VMEM((NGS, GS), _i32),
        pltpu.VMEM((NGS, GS, 16), _f32),
        pltpu.VMEM((NGS, GS, 16), _f32),
        pltpu.SemaphoreType.DMA,
    ],
)
def _sc_dualg(tab1, tab2, idx1, idx2, o1, o2, i1_v, i2_v, a_v, b_v, sem):
    cid = lax.axis_index("c")
    sid = lax.axis_index("s")
    w = cid * NS + sid
    cnt = (NCHUNK - w + NW - 1) // NW

    def chunk(kk, _):
        c = w + kk * NW
        r5 = pl.ds(c * NGS, NGS)
        _drain([pltpu.async_copy(idx1.at[r5], i1_v, sem),
                pltpu.async_copy(idx2.at[r5], i2_v, sem)])
        ds = []
        for g in range(NGS):
            ds.append(pltpu.async_copy(tab1.at[i1_v.at[g]], a_v.at[g], sem))
            ds.append(pltpu.async_copy(tab2.at[i2_v.at[g]], b_v.at[g], sem))
        _drain(ds)
        _drain([pltpu.async_copy(a_v, o1.at[r5], sem),
                pltpu.async_copy(b_v, o2.at[r5], sem)])
        return 0

    lax.fori_loop(0, cnt, chunk, 0)


@functools.partial(
    pl.kernel,
    out_type=[jax.ShapeDtypeStruct((N, 16), _f32),
              jax.ShapeDtypeStruct((N, 16), _f32)],
    mesh=_mesh,
    compiler_params=pltpu.CompilerParams(use_tc_tiling_on_sc=False),
    scratch_types=[
        pltpu.VMEM((NGS, GS), _i32),
        pltpu.VMEM((NGS, GS, 16), _f32),
        pltpu.VMEM_SHARED((N, 16), _f32),
        pltpu.SemaphoreType.DMA,
    ],
)
def _sc_sscat(va3, dst2, z16, a_o, b_o, idxd_v, b1_v, sh2, sem):
    cid = lax.axis_index("c")
    sid = lax.axis_index("s")
    rows = pl.ds(sid * RPW, RPW)
    pltpu.sync_copy(z16.at[rows], sh2.at[rows])
    plsc.subcore_barrier()

    w = cid * NS + sid
    cnt = (NCHUNK - w + NW - 1) // NW

    def chunk(k, _):
        c = w + k * NW
        r5 = pl.ds(c * NGS, NGS)
        _drain([pltpu.async_copy(dst2.at[r5], idxd_v, sem),
                pltpu.async_copy(va3.at[r5], b1_v, sem)])
        _drain([pltpu.async_copy(b1_v.at[g], sh2.at[idxd_v.at[g]], sem,
                                 add=True) for g in range(NGS)])
        return 0

    lax.fori_loop(0, cnt, chunk, 0)
    plsc.subcore_barrier()

    @pl.when(cid == 0)
    def _():
        pltpu.sync_copy(sh2.at[rows], a_o.at[rows])

    @pl.when(cid == 1)
    def _():
        pltpu.sync_copy(sh2.at[rows], b_o.at[rows])


@functools.partial(
    pl.kernel,
    out_type=[jax.ShapeDtypeStruct((N, 16), _f32),
              jax.ShapeDtypeStruct((N, 16), _f32)],
    mesh=_mesh,
    compiler_params=pltpu.CompilerParams(use_tc_tiling_on_sc=False),
    scratch_types=[
        pltpu.VMEM((NGS, GS), _i32),
        pltpu.VMEM((NGS, GS), _i32),
        pltpu.VMEM((NGS, GS, 16), _f32),
        pltpu.VMEM((NGS, GS, 16), _f32),
        pltpu.VMEM((NGS, GS, 16), _f32),
        pltpu.VMEM_SHARED((N, 16), _f32),
        pltpu.SemaphoreType.DMA,
    ],
)
def _sc_b1(t16, exrv3, src2, dst2, z16,
           sacc_o, dup_o,
           idxs_v, idxd_v, gt_v, p1_v, b1_v, sh2, sem):
    cid = lax.axis_index("c")
    sid = lax.axis_index("s")
    rows = pl.ds(sid * RPW, RPW)
    pltpu.sync_copy(z16.at[rows], sh2.at[rows])
    plsc.subcore_barrier()

    w = cid * NS + sid
    cnt = (NCHUNK - w + NW - 1) // NW

    def chunk(k, _):
        c = w + k * NW
        r5 = pl.ds(c * NGS, NGS)
        _drain([pltpu.async_copy(src2.at[r5], idxs_v, sem),
                pltpu.async_copy(dst2.at[r5], idxd_v, sem),
                pltpu.async_copy(exrv3.at[r5], p1_v, sem)])
        ds = [pltpu.async_copy(t16.at[idxs_v.at[g]], gt_v.at[g], sem)
              for g in range(NGS)]
        _drain(ds)
        for g in range(NGS):
            @plsc.parallel_loop(0, GS, unroll=8)
            def _(i):
                b1_v[g, i, :] = gt_v[g, i, :] * p1_v[g, i, :]
        _drain([pltpu.async_copy(b1_v.at[g], sh2.at[idxd_v.at[g]], sem,
                                 add=True) for g in range(NGS)])
        return 0

    lax.fori_loop(0, cnt, chunk, 0)
    plsc.subcore_barrier()

    @pl.when(cid == 0)
    def _():
        pltpu.sync_copy(sh2.at[rows], sacc_o.at[rows])

    @pl.when(cid == 1)
    def _():
        pltpu.sync_copy(sh2.at[rows], dup_o.at[rows])


@functools.partial(
    pl.kernel,
    out_type=[jax.ShapeDtypeStruct((N, 16), _f32),
              jax.ShapeDtypeStruct((N, 16), _f32)],
    mesh=_mesh,
    compiler_params=pltpu.CompilerParams(use_tc_tiling_on_sc=False),
    scratch_types=[
        pltpu.VMEM((NGS, GS), _i32),
        pltpu.VMEM((NGS, GS), _i32),
        pltpu.VMEM((NGS, GS, 32), _f32),
        pltpu.VMEM((NGS, GS, 16), _f32),
        pltpu.VMEM((NGS, GS, 16), _f32),
        pltpu.VMEM((NGS, GS, 16), _f32),
        pltpu.VMEM_SHARED((N, 16), _f32),
        pltpu.SemaphoreType.DMA,
    ],
)
def _sc_bv(t32, ex3, exu3, src2, dst2, z16,
           pa_o, pb_o,
           idxs_v, idxd_v, gt_v, p1_v, p2_v, b1_v, sh2, sem):
    cid = lax.axis_index("c")
    sid = lax.axis_index("s")
    rows = pl.ds(sid * RPW, RPW)
    pltpu.sync_copy(z16.at[rows], sh2.at[rows])
    plsc.subcore_barrier()

    w = cid * NS + sid
    cnt = (NCHUNK - w + NW - 1) // NW

    def chunk(k, _):
        c = w + k * NW
        r5 = pl.ds(c * NGS, NGS)
        _drain([pltpu.async_copy(src2.at[r5], idxs_v, sem),
                pltpu.async_copy(dst2.at[r5], idxd_v, sem),
                pltpu.async_copy(ex3.at[r5], p1_v, sem),
                pltpu.async_copy(exu3.at[r5], p2_v, sem)])
        ds = [pltpu.async_copy(t32.at[idxs_v.at[g]], gt_v.at[g], sem)
              for g in range(NGS)]
        _drain(ds)
        for g in range(NGS):
            @plsc.parallel_loop(0, GS, unroll=8)
            def _(i):
                b1_v[g, i, :] = (gt_v[g, i, 0:16] * p1_v[g, i, :]
                                 + gt_v[g, i, 16:32] * p2_v[g, i, :])
        _drain([pltpu.async_copy(b1_v.at[g], sh2.at[idxd_v.at[g]], sem,
                                 add=True) for g in range(NGS)])
        return 0

    lax.fori_loop(0, cnt, chunk, 0)
    plsc.subcore_barrier()

    @pl.when(cid == 0)
    def _():
        pltpu.sync_copy(sh2.at[rows], pa_o.at[rows])

    @pl.when(cid == 1)
    def _():
        pltpu.sync_copy(sh2.at[rows], pb_o.at[rows])


def kernel(node_types, pos, edge_index, batch, W_embed, b_embed, W_enc_s,
           b_enc_s, W_enc_r, Wq, Wk, Wrk, Wvs, Wrv, Wvv, Wsv, W_dec):
    del batch
    src2 = edge_index[0].reshape(NROW, GS)
    dst2 = edge_index[1].reshape(NROW, GS)
    pos16 = jnp.concatenate([pos, jnp.zeros((N, 13), _f32)], axis=1)
    z16 = jnp.zeros((N, 16), _f32)

    eye8 = jnp.eye(8, dtype=_f32)
    bdsum = jnp.kron(eye8, jnp.ones((16, 16), _f32))
    pad8 = jnp.zeros((8, 16), _f32)
    bdenc = jnp.kron(eye8, jnp.concatenate([W_enc_r, pad8], axis=0))
    bdrk = [jnp.kron(eye8, jnp.concatenate([Wrk[l], pad8], axis=0))
            for l in range(2)]
    bdrv = [jnp.kron(eye8, jnp.concatenate([Wrv[l], pad8], axis=0))
            for l in range(2)]
    bdp = [jnp.kron(eye8, jnp.zeros((16, 16), _f32).at[d].set(1.0))
           for d in range(3)]

    r3 = lambda a: a.reshape(NROW, GS, 16)
    rp = lambda a: a.reshape(EPR, 128)

    s = _tc_encode(node_types, W_embed, b_embed, W_enc_s, b_enc_s)

    gs3, gd3 = _sc_dualg(pos16, pos16, src2, dst2)
    uP, wu0, wu1, wu2, rk0, rk1, rv0, rv1 = _tc_geom(
        rp(gs3), rp(gd3), bdsum, bdenc, bdrk[0], bdrk[1], bdrv[0], bdrv[1],
        bdp[0], bdp[1], bdp[2])
    rk = (rk0, rk1)
    rv = (rv0, rv1)

    w0p = _sc_sscat(r3(wu0), dst2, z16)
    w1p = _sc_sscat(r3(wu1), dst2, z16)
    w2p = _sc_sscat(r3(wu2), dst2, z16)
    v0, v1, v2 = _tc_add3(w0p[0], w0p[1], w1p[0], w1p[1], w2p[0], w2p[1])

    for l in range(2):
        qt, kt, t16, t0, t1, t2 = _tc_tab(s, v0, v1, v2, Wq[l], Wk[l],
                                          Wvs[l], Wsv[l], Wvv[l])
        qd3, ks3 = _sc_dualg(qt, kt, dst2, src2)
        lgP, mx = _tc_logits(rp(qd3), rp(ks3), rk[l], bdsum)
        exP, exrv, exu0, exu1, exu2 = _tc_soft(lgP, mx, uP, rv[l],
                                               bdp[0], bdp[1], bdp[2])
        denp = _sc_sscat(r3(exP), dst2, z16)
        vap = [_sc_bv(t, r3(exP), r3(exu), src2, dst2, z16)
               for t, exu in ((t0, exu0), (t1, exu1), (t2, exu2))]
        if l == 0:
            sacc_a, sacc_b = _sc_b1(t16, r3(exrv), src2, dst2, z16)
            s, v0, v1, v2 = _tc_upd(s, v0, v1, v2, sacc_a, sacc_b, vap,
                                    denp)
        else:
            return _tc_out(v0, v1, v2, vap, denp, W_dec)
